# baseline (speedup 1.0000x reference)
import functools

import jax
import jax.numpy as jnp
from jax import lax
from jax.experimental import pallas as pl
from jax.experimental.pallas import tpu as pltpu
from jax.experimental.pallas import tpu_sc as plsc

EMBED = 128
NUM_WORKERS = 32
CHUNK = 400
GROUPS = CHUNK // 16


def _make_emb_kernel(n_rows: int, vocab: int):
    rows_per_w = n_rows // NUM_WORKERS
    n_chunks = rows_per_w // CHUNK
    n_pairs = n_chunks // 2
    chunk_elems = CHUNK * EMBED
    mesh = plsc.VectorSubcoreMesh(core_axis_name="c", subcore_axis_name="s")

    @functools.partial(
        pl.kernel,
        mesh=mesh,
        compiler_params=pltpu.CompilerParams(needs_layout_passes=False),
        out_type=jax.ShapeDtypeStruct((n_rows * EMBED,), jnp.float32),
        scratch_types=[
            pltpu.VMEM((vocab * EMBED,), jnp.float32),
            pltpu.VMEM((CHUNK,), jnp.int32),
            pltpu.VMEM((CHUNK,), jnp.int32),
            pltpu.VMEM((chunk_elems,), jnp.float32),
            pltpu.VMEM((chunk_elems,), jnp.float32),
            pltpu.SemaphoreType.DMA,
            pltpu.SemaphoreType.DMA,
            pltpu.SemaphoreType.DMA,
            pltpu.SemaphoreType.DMA,
        ],
    )
    def emb(ids_hbm, table_hbm, out_hbm, table_v, ids0, ids1, rows0, rows1,
            isem0, isem1, osem0, osem1):
        wid = lax.axis_index("s") * 2 + lax.axis_index("c")
        base = wid * rows_per_w
        last = n_chunks - 1

        pltpu.sync_copy(table_hbm, table_v)

        def ids_start(c, idsb, sem):
            c = jnp.minimum(c, last)
            pltpu.async_copy(
                ids_hbm.at[pl.ds(base + c * CHUNK, CHUNK)], idsb, sem)

        def ids_wait(c, idsb, sem):
            c = jnp.minimum(c, last)
            pltpu.make_async_copy(
                ids_hbm.at[pl.ds(base + c * CHUNK, CHUNK)], idsb, sem).wait()

        def compute(idsb, rows_v):
            @plsc.parallel_loop(0, GROUPS, unroll=5)
            def grp(g):
                v_t = idsb[pl.ds(g * 16, 16)] * EMBED
                tbs = [v_t[u] for u in range(16)]
                gb = g * (16 * EMBED)
                for u in range(16):
                    rb = gb + u * EMBED
                    vals = [table_v[pl.ds(tbs[u] + 16 * j, 16)]
                            for j in range(EMBED // 16)]
                    for j in range(EMBED // 16):
                        rows_v[pl.ds(rb + 16 * j, 16)] = vals[j]

        def out_start(c, rows_v, sem):
            pltpu.async_copy(
                rows_v,
                out_hbm.at[pl.ds((base + c * CHUNK) * EMBED, chunk_elems)],
                sem)

        def out_wait(c, rows_v, sem):
            pltpu.make_async_copy(
                rows_v,
                out_hbm.at[pl.ds((base + c * CHUNK) * EMBED, chunk_elems)],
                sem).wait()

        ids_start(0, ids0, isem0)
        ids_start(1, ids1, isem1)
        ids_wait(0, ids0, isem0)
        compute(ids0, rows0)
        ids_start(2, ids0, isem0)
        out_start(0, rows0, osem0)
        ids_wait(1, ids1, isem1)
        compute(ids1, rows1)
        ids_start(3, ids1, isem1)
        out_start(1, rows1, osem1)

        def body(j, carry):
            c0 = 2 * j
            c1 = c0 + 1
            out_wait(c0 - 2, rows0, osem0)
            ids_wait(c0, ids0, isem0)
            compute(ids0, rows0)
            ids_start(c0 + 2, ids0, isem0)
            out_start(c0, rows0, osem0)
            out_wait(c1 - 2, rows1, osem1)
            ids_wait(c1, ids1, isem1)
            compute(ids1, rows1)
            ids_start(c1 + 2, ids1, isem1)
            out_start(c1, rows1, osem1)
            return carry

        lax.fori_loop(1, n_pairs, body, 0)
        ids_wait(last, ids0, isem0)
        ids_wait(last, ids1, isem1)
        out_wait(n_chunks - 2, rows0, osem0)
        out_wait(last, rows1, osem1)

    return emb


def kernel(type_ids, lexicon_embeds):
    batch, seq = type_ids.shape
    vocab, embed = lexicon_embeds.shape
    n_rows = batch * seq
    ids = type_ids.reshape(n_rows).astype(jnp.int32)
    table = lexicon_embeds.reshape(vocab * embed)
    out = _make_emb_kernel(n_rows, vocab)(ids, table)
    return out.reshape(batch, seq, embed)


# speedup vs baseline: 1.0909x; 1.0909x over previous
import functools

import jax
import jax.numpy as jnp
from jax import lax
from jax.experimental import pallas as pl
from jax.experimental.pallas import tpu as pltpu
from jax.experimental.pallas import tpu_sc as plsc

EMBED = 128
NUM_WORKERS = 32
CHUNK = 320
GROUPS = CHUNK // 16


def _make_emb_kernel(n_rows: int, vocab: int):
    rows_per_w = n_rows // NUM_WORKERS
    n_chunks = rows_per_w // CHUNK
    n_pairs = n_chunks // 2
    chunk_elems = CHUNK * EMBED
    mesh = plsc.VectorSubcoreMesh(core_axis_name="c", subcore_axis_name="s")

    @functools.partial(
        pl.kernel,
        mesh=mesh,
        compiler_params=pltpu.CompilerParams(needs_layout_passes=False),
        out_type=jax.ShapeDtypeStruct((n_rows * EMBED,), jnp.float32),
        scratch_types=[
            pltpu.VMEM((vocab * EMBED,), jnp.float32),
            pltpu.VMEM((rows_per_w,), jnp.int32),
            pltpu.VMEM((chunk_elems,), jnp.float32),
            pltpu.VMEM((chunk_elems,), jnp.float32),
            pltpu.SemaphoreType.DMA,
            pltpu.SemaphoreType.DMA,
        ],
    )
    def emb(ids_hbm, table_hbm, out_hbm, table_v, ids_v, rows0, rows1,
            osem0, osem1):
        wid = lax.axis_index("s") * 2 + lax.axis_index("c")
        base = wid * rows_per_w

        pltpu.sync_copy(table_hbm, table_v)
        pltpu.sync_copy(ids_hbm.at[pl.ds(base, rows_per_w)], ids_v)

        def compute(c, rows_v):
            coff = c * CHUNK

            @plsc.parallel_loop(0, GROUPS, unroll=2)
            def grp(g):
                v_t = ids_v[pl.ds(coff + g * 16, 16)] * EMBED
                tbs = [v_t[u] for u in range(16)]
                gb = g * (16 * EMBED)
                for u in range(16):
                    rb = gb + u * EMBED
                    vals = [table_v[pl.ds(tbs[u] + 16 * j, 16)]
                            for j in range(EMBED // 16)]
                    for j in range(EMBED // 16):
                        rows_v[pl.ds(rb + 16 * j, 16)] = vals[j]

        def out_start(c, rows_v, sem):
            pltpu.async_copy(
                rows_v,
                out_hbm.at[pl.ds((base + c * CHUNK) * EMBED, chunk_elems)],
                sem)

        def out_wait(c, rows_v, sem):
            pltpu.make_async_copy(
                rows_v,
                out_hbm.at[pl.ds((base + c * CHUNK) * EMBED, chunk_elems)],
                sem).wait()

        compute(0, rows0)
        out_start(0, rows0, osem0)
        compute(1, rows1)
        out_start(1, rows1, osem1)

        def body(j, carry):
            c0 = 2 * j
            c1 = c0 + 1
            out_wait(c0 - 2, rows0, osem0)
            compute(c0, rows0)
            out_start(c0, rows0, osem0)
            out_wait(c1 - 2, rows1, osem1)
            compute(c1, rows1)
            out_start(c1, rows1, osem1)
            return carry

        lax.fori_loop(1, n_pairs, body, 0)
        out_wait(n_chunks - 2, rows0, osem0)
        out_wait(n_chunks - 1, rows1, osem1)

    return emb


def kernel(type_ids, lexicon_embeds):
    batch, seq = type_ids.shape
    vocab, embed = lexicon_embeds.shape
    n_rows = batch * seq
    ids = type_ids.reshape(n_rows).astype(jnp.int32)
    table = lexicon_embeds.reshape(vocab * embed)
    out = _make_emb_kernel(n_rows, vocab)(ids, table)
    return out.reshape(batch, seq, embed)
